# baseline (speedup 1.0000x reference)
import functools

import jax
import jax.numpy as jnp
from jax import lax
from jax.experimental import pallas as pl
from jax.experimental.pallas import tpu as pltpu
from jax.experimental.pallas import tpu_sc as plsc

N = 10000
E = 320000
CH = 128
H = 8
D = 16
AW = 32
BW = 16
PW = 144
NBLK = 10
RB = N // NBLK

NSUB = 16
EPT = E // NSUB
CHUNK = 80
NCHUNK = EPT // CHUNK
ACCN = 10240
RPT = ACCN // NSUB
ZR = 32


def _leaky(t):
    return jnp.where(t >= 0, t, 0.2 * t)


def _prep_body(x_ref, w_ref, att0_ref, att1_ref,
               a_ref, b_ref, gmax_ref, gs_src, gs_dst):
    i = pl.program_id(0)
    hb = lax.dot_general(x_ref[...], w_ref[...], (((1,), (1,)), ((), ())),
                         preferred_element_type=jnp.float32)
    rows_s = []
    rows_d = []
    for m, att_ref in enumerate((att0_ref, att1_ref)):
        att = att_ref[...]
        a_dst = att[:, :D]
        a_src = att[:, D:]
        ss = lax.dot_general(hb, a_src, (((1,), (1,)), ((), ())),
                             preferred_element_type=jnp.float32)
        sd = lax.dot_general(hb, a_dst, (((1,), (1,)), ((), ())),
                             preferred_element_type=jnp.float32)
        a_ref[m] = jnp.concatenate([hb, ss, ss], axis=-1)
        b_ref[m] = jnp.concatenate([sd, sd], axis=-1)
        ms = jnp.max(ss, axis=0)
        md = jnp.max(sd, axis=0)
        rows_s.append(jnp.concatenate([ms, ms]))
        rows_d.append(jnp.concatenate([md, md]))
    sstack = jnp.stack(rows_s)
    dstack = jnp.stack(rows_d)

    @pl.when(i == 0)
    def _():
        gs_src[...] = sstack
        gs_dst[...] = dstack

    @pl.when(i > 0)
    def _():
        gs_src[...] = jnp.maximum(gs_src[...], sstack)
        gs_dst[...] = jnp.maximum(gs_dst[...], dstack)

    @pl.when(i == NBLK - 1)
    def _():
        gmax_ref[...] = _leaky(gs_src[...] + gs_dst[...])


def _prep(x, w, att0, att1):
    return pl.pallas_call(
        _prep_body,
        grid=(NBLK,),
        in_specs=[
            pl.BlockSpec((RB, CH), lambda i: (i, 0)),
            pl.BlockSpec((D, CH), lambda i: (0, 0)),
            pl.BlockSpec((H, 2 * D), lambda i: (0, 0)),
            pl.BlockSpec((H, 2 * D), lambda i: (0, 0)),
        ],
        out_specs=[
            pl.BlockSpec((2, RB, AW), lambda i: (0, i, 0)),
            pl.BlockSpec((2, RB, BW), lambda i: (0, i, 0)),
            pl.BlockSpec((2, 16), lambda i: (0, 0)),
        ],
        out_shape=[
            jax.ShapeDtypeStruct((2, N, AW), jnp.float32),
            jax.ShapeDtypeStruct((2, N, BW), jnp.float32),
            jax.ShapeDtypeStruct((2, 16), jnp.float32),
        ],
        scratch_shapes=[
            pltpu.VMEM((2, 16), jnp.float32),
            pltpu.VMEM((2, 16), jnp.float32),
        ],
    )(x, w, att0, att1)


def _edge_body(a_hbm, b_hbm, idx_hbm, gmax_hbm, out_hbm,
               idx_all, idx_sc, a_rows, b_rows, payload,
               gmax_v, zbuf, acc, sem_i, sem_a, sem_b, sem_p):
    c = lax.axis_index("c")
    s = lax.axis_index("s")

    pltpu.sync_copy(gmax_hbm.at[c], gmax_v)
    gv = gmax_v[...]

    def _zrow(r, _):
        for k in range(PW // 16):
            zbuf[r, pl.ds(k * 16, 16)] = jnp.zeros((16,), jnp.float32)
        return 0
    lax.fori_loop(0, ZR, _zrow, 0)
    for j in range(RPT // ZR):
        pltpu.sync_copy(zbuf, acc.at[pl.ds(s * RPT + j * ZR, ZR)])
    plsc.subcore_barrier()

    rbase = (c * NSUB + s) * NCHUNK
    co = c * ACCN
    last = NCHUNK - 1

    def _issue_idx(j, b):
        row = rbase + jnp.minimum(j, last)
        pltpu.async_copy(idx_hbm.at[row], idx_all.at[b], sem_i.at[b])

    def _wait_idx(b):
        pltpu.make_async_copy(idx_hbm.at[0], idx_all.at[b],
                              sem_i.at[b]).wait()

    def _gather(b):
        pltpu.async_copy(a_hbm.at[idx_all.at[b, 0]], a_rows.at[b],
                         sem_a.at[b])
        pltpu.async_copy(b_hbm.at[idx_all.at[b, 1]], b_rows.at[b],
                         sem_b.at[b])

    def _wait_gather(b):
        pltpu.make_async_copy(a_hbm.at[idx_all.at[b, 0]], a_rows.at[b],
                              sem_a.at[b]).wait()
        pltpu.make_async_copy(b_hbm.at[idx_all.at[b, 1]], b_rows.at[b],
                              sem_b.at[b]).wait()

    def _compute(b):
        @plsc.parallel_loop(0, CHUNK, 1, unroll=8)
        def _edge(e):
            va = a_rows[b, e, pl.ds(0, 16)]
            vs = a_rows[b, e, pl.ds(16, 16)]
            vb = b_rows[b, e, pl.ds(0, 16)]
            ex = jnp.exp(_leaky(vs + vb) - gv)
            payload[b, e, pl.ds(CH, 16)] = ex
            for hh in range(H):
                w = ex.at[jnp.full((16,), hh, jnp.int32)].get(
                    mode="promise_in_bounds")
                payload[b, e, pl.ds(hh * 16, 16)] = w * va

    def _wait_scatter(b):
        pltpu.make_async_copy(payload.at[b], acc.at[idx_sc.at[b]],
                              sem_p.at[b]).wait()

    _issue_idx(0, 0)
    _wait_idx(0)
    _gather(0)
    _issue_idx(1, 1)

    def _two_chunks(j2, _):
        j = 2 * j2
        for b in (0, 1):
            nb = 1 - b
            _wait_gather(b)
            _wait_idx(nb)
            _gather(nb)

            @pl.when(j2 > 0)
            def _():
                _wait_scatter(b)
            for k in range(CHUNK // 16):
                sl = pl.ds(k * 16, 16)
                idx_sc[b, sl] = idx_all[b, 2, sl]
            _issue_idx(j + b + 2, b)
            _compute(b)
            pltpu.async_copy(payload.at[b], acc.at[idx_sc.at[b]],
                             sem_p.at[b], add=True)
        return 0

    lax.fori_loop(0, NCHUNK // 2, _two_chunks, 0)
    _wait_gather(0)
    _wait_idx(1)
    _wait_scatter(0)
    _wait_scatter(1)
    plsc.subcore_barrier()

    for j in range(RPT // ZR):
        r = s * RPT + j * ZR
        pltpu.sync_copy(acc.at[pl.ds(r, ZR)], out_hbm.at[pl.ds(co + r, ZR)])


def _edge_phase(a_tab, b_tab, idx_tab, gmax):
    f = pl.kernel(
        _edge_body,
        out_type=jax.ShapeDtypeStruct((2 * ACCN, PW), jnp.float32),
        mesh=plsc.VectorSubcoreMesh(core_axis_name="c", subcore_axis_name="s"),
        scratch_types=[
            pltpu.VMEM((2, 3, CHUNK), jnp.int32),
            pltpu.VMEM((2, CHUNK), jnp.int32),
            pltpu.VMEM((2, CHUNK, AW), jnp.float32),
            pltpu.VMEM((2, CHUNK, BW), jnp.float32),
            pltpu.VMEM((2, CHUNK, PW), jnp.float32),
            pltpu.VMEM((16,), jnp.float32),
            pltpu.VMEM((ZR, PW), jnp.float32),
            pltpu.VMEM_SHARED((ACCN, PW), jnp.float32),
            pltpu.SemaphoreType.DMA((2,)),
            pltpu.SemaphoreType.DMA((2,)),
            pltpu.SemaphoreType.DMA((2,)),
            pltpu.SemaphoreType.DMA((2,)),
        ],
        compiler_params=pltpu.CompilerParams(use_tc_tiling_on_sc=False),
    )
    return f(a_tab, b_tab, idx_tab, gmax)


def _finish_body(p_ref, a_ref, b_ref, gmax_ref, fcw_ref, fcb_ref, q_ref,
                 z_ref, sc_ref):
    i = pl.program_id(0)
    parts = []
    for m in range(2):
        pm = p_ref[m]
        accf = pm[:, :CH]
        den = pm[:, CH:CH + H]
        hb = a_ref[m][:, :D]
        ss = a_ref[m][:, D:D + H]
        sd = b_ref[m][:, :H]
        g = gmax_ref[...][m, :H]
        exs = jnp.exp(_leaky(ss + sd) - g[None, :])
        den2 = den + exs + 1e-16
        exw = jnp.repeat(exs, D, axis=1)
        denw = jnp.repeat(den2, D, axis=1)
        hw = jnp.tile(hb, (1, H))
        out = (accf + exw * hw) / denw
        elu = jnp.where(out > 0, out, jnp.exp(jnp.minimum(out, 0.0)) - 1.0)
        z_ref[m] = elu
        t = jnp.tanh(lax.dot_general(elu, fcw_ref[...],
                                     (((1,), (1,)), ((), ())),
                                     preferred_element_type=jnp.float32)
                     + fcb_ref[...])
        parts.append(jnp.sum(t * q_ref[...]) * (1.0 / N))
    lane = lax.broadcasted_iota(jnp.int32, (1, CH), 1)
    srow = jnp.where(lane == 0, parts[0], jnp.where(lane == 1, parts[1], 0.0))

    @pl.when(i == 0)
    def _():
        sc_ref[...] = srow

    @pl.when(i > 0)
    def _():
        sc_ref[...] = sc_ref[...] + srow


def _finish(p3, a_tab3, b_tab3, gmax, fc_w, fc_b, q):
    return pl.pallas_call(
        _finish_body,
        grid=(NBLK,),
        in_specs=[
            pl.BlockSpec((2, RB, PW), lambda i: (0, i, 0)),
            pl.BlockSpec((2, RB, AW), lambda i: (0, i, 0)),
            pl.BlockSpec((2, RB, BW), lambda i: (0, i, 0)),
            pl.BlockSpec((2, 16), lambda i: (0, 0)),
            pl.BlockSpec((CH, CH), lambda i: (0, 0)),
            pl.BlockSpec((1, CH), lambda i: (0, 0)),
            pl.BlockSpec((1, CH), lambda i: (0, 0)),
        ],
        out_specs=[
            pl.BlockSpec((2, RB, CH), lambda i: (0, i, 0)),
            pl.BlockSpec((1, CH), lambda i: (0, 0)),
        ],
        out_shape=[
            jax.ShapeDtypeStruct((2, N, CH), jnp.float32),
            jax.ShapeDtypeStruct((1, CH), jnp.float32),
        ],
    )(p3, a_tab3, b_tab3, gmax, fc_w, fc_b, q)


def _blend_body(z_ref, sc_ref, o_ref):
    srow = sc_ref[...]
    lane = lax.broadcasted_iota(jnp.int32, (1, CH), 1)
    s0 = jnp.sum(jnp.where(lane == 0, srow, 0.0))
    s1 = jnp.sum(jnp.where(lane == 1, srow, 0.0))
    mx = jnp.maximum(s0, s1)
    e0 = jnp.exp(s0 - mx)
    e1 = jnp.exp(s1 - mx)
    v0 = e0 / (e0 + e1)
    o_ref[...] = v0 * z_ref[0] + (1.0 - v0) * z_ref[1]


def _blend(z, scores):
    return pl.pallas_call(
        _blend_body,
        grid=(NBLK,),
        in_specs=[
            pl.BlockSpec((2, RB, CH), lambda i: (0, i, 0)),
            pl.BlockSpec((1, CH), lambda i: (0, 0)),
        ],
        out_specs=pl.BlockSpec((RB, CH), lambda i: (i, 0)),
        out_shape=jax.ShapeDtypeStruct((N, CH), jnp.float32),
    )(z, scores)


def kernel(x_paper, edge_index_cites, edge_index_refs, W_proj, att0, att1,
           fc_w, fc_b, q):
    a_tab3, b_tab3, gmax = _prep(x_paper, W_proj, att0, att1)
    a_tab = a_tab3.reshape(2 * N, AW)
    b_tab = b_tab3.reshape(2 * N, BW)
    ei = jnp.stack([edge_index_cites, edge_index_refs])
    off = jnp.array([0, N], jnp.int32).reshape(2, 1, 1)
    idx3 = jnp.stack([ei[:, 0] + off[..., 0], ei[:, 1] + off[..., 0],
                      ei[:, 1]], axis=1)
    idx_tab = (idx3.reshape(2, 3, NSUB, NCHUNK, CHUNK)
               .transpose(0, 2, 3, 1, 4)
               .reshape(2 * NSUB * NCHUNK, 3, CHUNK))
    p = _edge_phase(a_tab, b_tab, idx_tab, gmax)
    p3 = p.reshape(2, ACCN, PW)[:, :N, :]
    z, scores = _finish(p3, a_tab3, b_tab3, gmax, fc_w,
                        fc_b.reshape(1, CH), q.reshape(1, CH))
    return _blend(z, scores)


# speedup vs baseline: 1.0075x; 1.0075x over previous
import functools

import jax
import jax.numpy as jnp
from jax import lax
from jax.experimental import pallas as pl
from jax.experimental.pallas import tpu as pltpu
from jax.experimental.pallas import tpu_sc as plsc

N = 10000
E = 320000
CH = 128
H = 8
D = 16
AW = 32
BW = 16
PW = 144
NBLK = 10
RB = N // NBLK

NSUB = 16
EPT = E // NSUB
CHUNK = 80
NCHUNK = EPT // CHUNK
ACCN = 10240
RPT = ACCN // NSUB
ZR = 32


def _leaky(t):
    return jnp.where(t >= 0, t, 0.2 * t)


def _prep_body(x_ref, w_ref, att0_ref, att1_ref,
               a_ref, b_ref, gmax_ref, gs_src, gs_dst):
    i = pl.program_id(0)
    hb = lax.dot_general(x_ref[...], w_ref[...], (((1,), (1,)), ((), ())),
                         preferred_element_type=jnp.float32)
    rows_s = []
    rows_d = []
    for m, att_ref in enumerate((att0_ref, att1_ref)):
        att = att_ref[...]
        a_dst = att[:, :D]
        a_src = att[:, D:]
        ss = lax.dot_general(hb, a_src, (((1,), (1,)), ((), ())),
                             preferred_element_type=jnp.float32)
        sd = lax.dot_general(hb, a_dst, (((1,), (1,)), ((), ())),
                             preferred_element_type=jnp.float32)
        a_ref[m] = jnp.concatenate([hb, ss, ss], axis=-1)
        b_ref[m] = jnp.concatenate([sd, sd], axis=-1)
        ms = jnp.max(ss, axis=0)
        md = jnp.max(sd, axis=0)
        rows_s.append(jnp.concatenate([ms, ms]))
        rows_d.append(jnp.concatenate([md, md]))
    sstack = jnp.stack(rows_s)
    dstack = jnp.stack(rows_d)

    @pl.when(i == 0)
    def _():
        gs_src[...] = sstack
        gs_dst[...] = dstack

    @pl.when(i > 0)
    def _():
        gs_src[...] = jnp.maximum(gs_src[...], sstack)
        gs_dst[...] = jnp.maximum(gs_dst[...], dstack)

    @pl.when(i == NBLK - 1)
    def _():
        gmax_ref[...] = _leaky(gs_src[...] + gs_dst[...])


def _prep(x, w, att0, att1):
    return pl.pallas_call(
        _prep_body,
        grid=(NBLK,),
        in_specs=[
            pl.BlockSpec((RB, CH), lambda i: (i, 0)),
            pl.BlockSpec((D, CH), lambda i: (0, 0)),
            pl.BlockSpec((H, 2 * D), lambda i: (0, 0)),
            pl.BlockSpec((H, 2 * D), lambda i: (0, 0)),
        ],
        out_specs=[
            pl.BlockSpec((2, RB, AW), lambda i: (0, i, 0)),
            pl.BlockSpec((2, RB, BW), lambda i: (0, i, 0)),
            pl.BlockSpec((2, 16), lambda i: (0, 0)),
        ],
        out_shape=[
            jax.ShapeDtypeStruct((2, N, AW), jnp.float32),
            jax.ShapeDtypeStruct((2, N, BW), jnp.float32),
            jax.ShapeDtypeStruct((2, 16), jnp.float32),
        ],
        scratch_shapes=[
            pltpu.VMEM((2, 16), jnp.float32),
            pltpu.VMEM((2, 16), jnp.float32),
        ],
    )(x, w, att0, att1)


def _edge_body(a_hbm, b_hbm, idx_hbm, gmax_hbm, out_hbm,
               idx_all, idx_sc, a_rows, b_rows, payload,
               gmax_v, zbuf, acc, sem_i, sem_a, sem_b, sem_p):
    c = lax.axis_index("c")
    s = lax.axis_index("s")

    pltpu.sync_copy(gmax_hbm.at[c], gmax_v)
    gv = gmax_v[...]

    def _zrow(r, _):
        for k in range(PW // 16):
            zbuf[r, pl.ds(k * 16, 16)] = jnp.zeros((16,), jnp.float32)
        return 0
    lax.fori_loop(0, ZR, _zrow, 0)
    for j in range(RPT // ZR):
        pltpu.sync_copy(zbuf, acc.at[pl.ds(s * RPT + j * ZR, ZR)])
    plsc.subcore_barrier()

    rbase = (c * NSUB + s) * NCHUNK
    co = c * ACCN
    last = NCHUNK - 1

    def _issue_idx(j, b):
        row = rbase + jnp.minimum(j, last)
        pltpu.async_copy(idx_hbm.at[row], idx_all.at[b], sem_i.at[b])

    def _wait_idx(b):
        pltpu.make_async_copy(idx_hbm.at[0], idx_all.at[b],
                              sem_i.at[b]).wait()

    def _gather(b):
        pltpu.async_copy(a_hbm.at[idx_all.at[b, 0]], a_rows.at[b],
                         sem_a.at[b])
        pltpu.async_copy(b_hbm.at[idx_all.at[b, 1]], b_rows.at[b],
                         sem_b.at[b])

    def _wait_gather(b):
        pltpu.make_async_copy(a_hbm.at[idx_all.at[b, 0]], a_rows.at[b],
                              sem_a.at[b]).wait()
        pltpu.make_async_copy(b_hbm.at[idx_all.at[b, 1]], b_rows.at[b],
                              sem_b.at[b]).wait()

    def _compute(b):
        @plsc.parallel_loop(0, CHUNK, 1, unroll=4)
        def _edge(e):
            va = a_rows[b, e, pl.ds(0, 16)]
            vs = a_rows[b, e, pl.ds(16, 16)]
            vb = b_rows[b, e, pl.ds(0, 16)]
            ex = jnp.exp(_leaky(vs + vb) - gv)
            payload[b, e, pl.ds(CH, 16)] = ex
            for hh in range(H):
                w = ex.at[jnp.full((16,), hh, jnp.int32)].get(
                    mode="promise_in_bounds")
                payload[b, e, pl.ds(hh * 16, 16)] = w * va

    def _wait_scatter(b):
        pltpu.make_async_copy(payload.at[b], acc.at[idx_sc.at[b]],
                              sem_p.at[b]).wait()

    _issue_idx(0, 0)
    _wait_idx(0)
    _gather(0)
    _issue_idx(1, 1)

    def _two_chunks(j2, _):
        j = 2 * j2
        for b in (0, 1):
            nb = 1 - b
            _wait_gather(b)
            _wait_idx(nb)
            _gather(nb)

            @pl.when(j2 > 0)
            def _():
                _wait_scatter(b)
            for k in range(CHUNK // 16):
                sl = pl.ds(k * 16, 16)
                idx_sc[b, sl] = idx_all[b, 2, sl]
            _issue_idx(j + b + 2, b)
            _compute(b)
            pltpu.async_copy(payload.at[b], acc.at[idx_sc.at[b]],
                             sem_p.at[b], add=True)
        return 0

    lax.fori_loop(0, NCHUNK // 2, _two_chunks, 0)
    _wait_gather(0)
    _wait_idx(1)
    _wait_scatter(0)
    _wait_scatter(1)
    plsc.subcore_barrier()

    for j in range(RPT // ZR):
        r = s * RPT + j * ZR
        pltpu.sync_copy(acc.at[pl.ds(r, ZR)], out_hbm.at[pl.ds(co + r, ZR)])


def _edge_phase(a_tab, b_tab, idx_tab, gmax):
    f = pl.kernel(
        _edge_body,
        out_type=jax.ShapeDtypeStruct((2 * ACCN, PW), jnp.float32),
        mesh=plsc.VectorSubcoreMesh(core_axis_name="c", subcore_axis_name="s"),
        scratch_types=[
            pltpu.VMEM((2, 3, CHUNK), jnp.int32),
            pltpu.VMEM((2, CHUNK), jnp.int32),
            pltpu.VMEM((2, CHUNK, AW), jnp.float32),
            pltpu.VMEM((2, CHUNK, BW), jnp.float32),
            pltpu.VMEM((2, CHUNK, PW), jnp.float32),
            pltpu.VMEM((16,), jnp.float32),
            pltpu.VMEM((ZR, PW), jnp.float32),
            pltpu.VMEM_SHARED((ACCN, PW), jnp.float32),
            pltpu.SemaphoreType.DMA((2,)),
            pltpu.SemaphoreType.DMA((2,)),
            pltpu.SemaphoreType.DMA((2,)),
            pltpu.SemaphoreType.DMA((2,)),
        ],
        compiler_params=pltpu.CompilerParams(use_tc_tiling_on_sc=False),
    )
    return f(a_tab, b_tab, idx_tab, gmax)


def _finish_body(p_ref, a_ref, b_ref, gmax_ref, fcw_ref, fcb_ref, q_ref,
                 z_ref, sc_ref):
    i = pl.program_id(0)
    parts = []
    for m in range(2):
        pm = p_ref[m]
        accf = pm[:, :CH]
        den = pm[:, CH:CH + H]
        hb = a_ref[m][:, :D]
        ss = a_ref[m][:, D:D + H]
        sd = b_ref[m][:, :H]
        g = gmax_ref[...][m, :H]
        exs = jnp.exp(_leaky(ss + sd) - g[None, :])
        den2 = den + exs + 1e-16
        exw = jnp.repeat(exs, D, axis=1)
        denw = jnp.repeat(den2, D, axis=1)
        hw = jnp.tile(hb, (1, H))
        out = (accf + exw * hw) / denw
        elu = jnp.where(out > 0, out, jnp.exp(jnp.minimum(out, 0.0)) - 1.0)
        z_ref[m] = elu
        t = jnp.tanh(lax.dot_general(elu, fcw_ref[...],
                                     (((1,), (1,)), ((), ())),
                                     preferred_element_type=jnp.float32)
                     + fcb_ref[...])
        parts.append(jnp.sum(t * q_ref[...]) * (1.0 / N))
    lane = lax.broadcasted_iota(jnp.int32, (1, CH), 1)
    srow = jnp.where(lane == 0, parts[0], jnp.where(lane == 1, parts[1], 0.0))

    @pl.when(i == 0)
    def _():
        sc_ref[...] = srow

    @pl.when(i > 0)
    def _():
        sc_ref[...] = sc_ref[...] + srow


def _finish(p3, a_tab3, b_tab3, gmax, fc_w, fc_b, q):
    return pl.pallas_call(
        _finish_body,
        grid=(NBLK,),
        in_specs=[
            pl.BlockSpec((2, RB, PW), lambda i: (0, i, 0)),
            pl.BlockSpec((2, RB, AW), lambda i: (0, i, 0)),
            pl.BlockSpec((2, RB, BW), lambda i: (0, i, 0)),
            pl.BlockSpec((2, 16), lambda i: (0, 0)),
            pl.BlockSpec((CH, CH), lambda i: (0, 0)),
            pl.BlockSpec((1, CH), lambda i: (0, 0)),
            pl.BlockSpec((1, CH), lambda i: (0, 0)),
        ],
        out_specs=[
            pl.BlockSpec((2, RB, CH), lambda i: (0, i, 0)),
            pl.BlockSpec((1, CH), lambda i: (0, 0)),
        ],
        out_shape=[
            jax.ShapeDtypeStruct((2, N, CH), jnp.float32),
            jax.ShapeDtypeStruct((1, CH), jnp.float32),
        ],
    )(p3, a_tab3, b_tab3, gmax, fc_w, fc_b, q)


def _blend_body(z_ref, sc_ref, o_ref):
    srow = sc_ref[...]
    lane = lax.broadcasted_iota(jnp.int32, (1, CH), 1)
    s0 = jnp.sum(jnp.where(lane == 0, srow, 0.0))
    s1 = jnp.sum(jnp.where(lane == 1, srow, 0.0))
    mx = jnp.maximum(s0, s1)
    e0 = jnp.exp(s0 - mx)
    e1 = jnp.exp(s1 - mx)
    v0 = e0 / (e0 + e1)
    o_ref[...] = v0 * z_ref[0] + (1.0 - v0) * z_ref[1]


def _blend(z, scores):
    return pl.pallas_call(
        _blend_body,
        grid=(NBLK,),
        in_specs=[
            pl.BlockSpec((2, RB, CH), lambda i: (0, i, 0)),
            pl.BlockSpec((1, CH), lambda i: (0, 0)),
        ],
        out_specs=pl.BlockSpec((RB, CH), lambda i: (i, 0)),
        out_shape=jax.ShapeDtypeStruct((N, CH), jnp.float32),
    )(z, scores)


def kernel(x_paper, edge_index_cites, edge_index_refs, W_proj, att0, att1,
           fc_w, fc_b, q):
    a_tab3, b_tab3, gmax = _prep(x_paper, W_proj, att0, att1)
    a_tab = a_tab3.reshape(2 * N, AW)
    b_tab = b_tab3.reshape(2 * N, BW)
    ei = jnp.stack([edge_index_cites, edge_index_refs])
    off = jnp.array([0, N], jnp.int32).reshape(2, 1, 1)
    idx3 = jnp.stack([ei[:, 0] + off[..., 0], ei[:, 1] + off[..., 0],
                      ei[:, 1]], axis=1)
    idx_tab = (idx3.reshape(2, 3, NSUB, NCHUNK, CHUNK)
               .transpose(0, 2, 3, 1, 4)
               .reshape(2 * NSUB * NCHUNK, 3, CHUNK))
    p = _edge_phase(a_tab, b_tab, idx_tab, gmax)
    p3 = p.reshape(2, ACCN, PW)[:, :N, :]
    z, scores = _finish(p3, a_tab3, b_tab3, gmax, fc_w,
                        fc_b.reshape(1, CH), q.reshape(1, CH))
    return _blend(z, scores)


# speedup vs baseline: 1.1676x; 1.1589x over previous
import functools

import jax
import jax.numpy as jnp
from jax import lax
from jax.experimental import pallas as pl
from jax.experimental.pallas import tpu as pltpu
from jax.experimental.pallas import tpu_sc as plsc

N = 10000
E = 320000
CH = 128
H = 8
D = 16
AW = 32
BW = 16
PW = 144
NBLK = 10
RB = N // NBLK

NSUB = 16
EPT = E // NSUB
CHUNK = 80
NCHUNK = EPT // CHUNK
ACCN = 10240
RPT = ACCN // NSUB
ZR = 32


def _leaky(t):
    return jnp.where(t >= 0, t, 0.2 * t)


def _prep_body(x_ref, w_ref, att0_ref, att1_ref,
               a_ref, b_ref, gmax_ref, gs_src, gs_dst):
    i = pl.program_id(0)
    hb = lax.dot_general(x_ref[...], w_ref[...], (((1,), (1,)), ((), ())),
                         preferred_element_type=jnp.float32)
    rows_s = []
    rows_d = []
    for m, att_ref in enumerate((att0_ref, att1_ref)):
        att = att_ref[...]
        a_dst = att[:, :D]
        a_src = att[:, D:]
        ss = lax.dot_general(hb, a_src, (((1,), (1,)), ((), ())),
                             preferred_element_type=jnp.float32)
        sd = lax.dot_general(hb, a_dst, (((1,), (1,)), ((), ())),
                             preferred_element_type=jnp.float32)
        a_ref[m] = jnp.concatenate([hb, ss, ss], axis=-1)
        b_ref[m] = jnp.concatenate([sd, sd], axis=-1)
        ms = jnp.max(ss, axis=0)
        md = jnp.max(sd, axis=0)
        rows_s.append(jnp.concatenate([ms, ms]))
        rows_d.append(jnp.concatenate([md, md]))
    sstack = jnp.stack(rows_s)
    dstack = jnp.stack(rows_d)

    @pl.when(i == 0)
    def _():
        gs_src[...] = sstack
        gs_dst[...] = dstack

    @pl.when(i > 0)
    def _():
        gs_src[...] = jnp.maximum(gs_src[...], sstack)
        gs_dst[...] = jnp.maximum(gs_dst[...], dstack)

    @pl.when(i == NBLK - 1)
    def _():
        gmax_ref[...] = _leaky(gs_src[...] + gs_dst[...])


def _prep(x, w, att0, att1):
    return pl.pallas_call(
        _prep_body,
        grid=(NBLK,),
        in_specs=[
            pl.BlockSpec((RB, CH), lambda i: (i, 0)),
            pl.BlockSpec((D, CH), lambda i: (0, 0)),
            pl.BlockSpec((H, 2 * D), lambda i: (0, 0)),
            pl.BlockSpec((H, 2 * D), lambda i: (0, 0)),
        ],
        out_specs=[
            pl.BlockSpec((2, RB, AW), lambda i: (0, i, 0)),
            pl.BlockSpec((2, RB, BW), lambda i: (0, i, 0)),
            pl.BlockSpec((2, 16), lambda i: (0, 0)),
        ],
        out_shape=[
            jax.ShapeDtypeStruct((2, N, AW), jnp.float32),
            jax.ShapeDtypeStruct((2, N, BW), jnp.float32),
            jax.ShapeDtypeStruct((2, 16), jnp.float32),
        ],
        scratch_shapes=[
            pltpu.VMEM((2, 16), jnp.float32),
            pltpu.VMEM((2, 16), jnp.float32),
        ],
    )(x, w, att0, att1)


def _edge_body(a_hbm, b_hbm, idx_hbm, gmax_hbm, out_hbm,
               idx_all, idx_sc, a_rows, b_rows, payload,
               gmax_v, zbuf, acc, sem_i, sem_a, sem_b, sem_p):
    c = lax.axis_index("c")
    s = lax.axis_index("s")

    pltpu.sync_copy(gmax_hbm.at[c], gmax_v)
    gv = gmax_v[...]

    def _zrow(r, _):
        for k in range(PW // 16):
            zbuf[r, pl.ds(k * 16, 16)] = jnp.zeros((16,), jnp.float32)
        return 0
    lax.fori_loop(0, ZR, _zrow, 0)
    for j in range(RPT // ZR):
        pltpu.sync_copy(zbuf, acc.at[pl.ds(s * RPT + j * ZR, ZR)])
    plsc.subcore_barrier()

    ebase = s * EPT
    co = c * ACCN
    last = NCHUNK - 1

    def _issue_idx(j, b):
        off = ebase + jnp.minimum(j, last) * CHUNK
        pltpu.async_copy(idx_hbm.at[c, :, pl.ds(off, CHUNK)], idx_all.at[b],
                         sem_i.at[b])

    def _wait_idx(b):
        pltpu.make_async_copy(idx_hbm.at[0, :, pl.ds(0, CHUNK)],
                              idx_all.at[b], sem_i.at[b]).wait()

    def _gather(b):
        pltpu.async_copy(a_hbm.at[idx_all.at[b, 0]], a_rows.at[b],
                         sem_a.at[b])
        pltpu.async_copy(b_hbm.at[idx_all.at[b, 1]], b_rows.at[b],
                         sem_b.at[b])

    def _wait_gather(b):
        pltpu.make_async_copy(a_hbm.at[idx_all.at[b, 0]], a_rows.at[b],
                              sem_a.at[b]).wait()
        pltpu.make_async_copy(b_hbm.at[idx_all.at[b, 1]], b_rows.at[b],
                              sem_b.at[b]).wait()

    def _compute(b):
        @plsc.parallel_loop(0, CHUNK, 1, unroll=4)
        def _edge(e):
            va = a_rows[b, e, pl.ds(0, 16)]
            vs = a_rows[b, e, pl.ds(16, 16)]
            vb = b_rows[b, e, pl.ds(0, 16)]
            ex = jnp.exp(_leaky(vs + vb) - gv)
            payload[b, e, pl.ds(CH, 16)] = ex
            for hh in range(H):
                w = ex.at[jnp.full((16,), hh, jnp.int32)].get(
                    mode="promise_in_bounds")
                payload[b, e, pl.ds(hh * 16, 16)] = w * va

    def _wait_scatter(b):
        pltpu.make_async_copy(payload.at[b], acc.at[idx_sc.at[b]],
                              sem_p.at[b]).wait()

    _issue_idx(0, 0)
    _wait_idx(0)
    _gather(0)
    _issue_idx(1, 1)

    def _two_chunks(j2, _):
        j = 2 * j2
        for b in (0, 1):
            nb = 1 - b
            _wait_gather(b)
            _wait_idx(nb)
            _gather(nb)

            @pl.when(j2 > 0)
            def _():
                _wait_scatter(b)
            for k in range(CHUNK // 16):
                sl = pl.ds(k * 16, 16)
                idx_sc[b, sl] = idx_all[b, 2, sl]
            _issue_idx(j + b + 2, b)
            _compute(b)
            pltpu.async_copy(payload.at[b], acc.at[idx_sc.at[b]],
                             sem_p.at[b], add=True)
        return 0

    lax.fori_loop(0, NCHUNK // 2, _two_chunks, 0)
    _wait_gather(0)
    _wait_idx(1)
    _wait_scatter(0)
    _wait_scatter(1)
    plsc.subcore_barrier()

    for j in range(RPT // ZR):
        r = s * RPT + j * ZR
        pltpu.sync_copy(acc.at[pl.ds(r, ZR)], out_hbm.at[pl.ds(co + r, ZR)])


def _edge_phase(a_tab, b_tab, idx_tab, gmax):
    f = pl.kernel(
        _edge_body,
        out_type=jax.ShapeDtypeStruct((2 * ACCN, PW), jnp.float32),
        mesh=plsc.VectorSubcoreMesh(core_axis_name="c", subcore_axis_name="s"),
        scratch_types=[
            pltpu.VMEM((2, 3, CHUNK), jnp.int32),
            pltpu.VMEM((2, CHUNK), jnp.int32),
            pltpu.VMEM((2, CHUNK, AW), jnp.float32),
            pltpu.VMEM((2, CHUNK, BW), jnp.float32),
            pltpu.VMEM((2, CHUNK, PW), jnp.float32),
            pltpu.VMEM((16,), jnp.float32),
            pltpu.VMEM((ZR, PW), jnp.float32),
            pltpu.VMEM_SHARED((ACCN, PW), jnp.float32),
            pltpu.SemaphoreType.DMA((2,)),
            pltpu.SemaphoreType.DMA((2,)),
            pltpu.SemaphoreType.DMA((2,)),
            pltpu.SemaphoreType.DMA((2,)),
        ],
        compiler_params=pltpu.CompilerParams(use_tc_tiling_on_sc=False),
    )
    return f(a_tab, b_tab, idx_tab, gmax)


def _finish_body(p_ref, a_ref, b_ref, gmax_ref, fcw_ref, fcb_ref, q_ref,
                 z_ref, sc_ref):
    i = pl.program_id(0)
    parts = []
    for m in range(2):
        pm = p_ref[m]
        accf = pm[:, :CH]
        den = pm[:, CH:CH + H]
        hb = a_ref[m][:, :D]
        ss = a_ref[m][:, D:D + H]
        sd = b_ref[m][:, :H]
        g = gmax_ref[...][m, :H]
        exs = jnp.exp(_leaky(ss + sd) - g[None, :])
        den2 = den + exs + 1e-16
        exw = jnp.repeat(exs, D, axis=1)
        denw = jnp.repeat(den2, D, axis=1)
        hw = jnp.tile(hb, (1, H))
        out = (accf + exw * hw) / denw
        elu = jnp.where(out > 0, out, jnp.exp(jnp.minimum(out, 0.0)) - 1.0)
        z_ref[m] = elu
        t = jnp.tanh(lax.dot_general(elu, fcw_ref[...],
                                     (((1,), (1,)), ((), ())),
                                     preferred_element_type=jnp.float32)
                     + fcb_ref[...])
        parts.append(jnp.sum(t * q_ref[...]) * (1.0 / N))
    lane = lax.broadcasted_iota(jnp.int32, (1, CH), 1)
    srow = jnp.where(lane == 0, parts[0], jnp.where(lane == 1, parts[1], 0.0))

    @pl.when(i == 0)
    def _():
        sc_ref[...] = srow

    @pl.when(i > 0)
    def _():
        sc_ref[...] = sc_ref[...] + srow


def _finish(p3, a_tab3, b_tab3, gmax, fc_w, fc_b, q):
    return pl.pallas_call(
        _finish_body,
        grid=(NBLK,),
        in_specs=[
            pl.BlockSpec((2, RB, PW), lambda i: (0, i, 0)),
            pl.BlockSpec((2, RB, AW), lambda i: (0, i, 0)),
            pl.BlockSpec((2, RB, BW), lambda i: (0, i, 0)),
            pl.BlockSpec((2, 16), lambda i: (0, 0)),
            pl.BlockSpec((CH, CH), lambda i: (0, 0)),
            pl.BlockSpec((1, CH), lambda i: (0, 0)),
            pl.BlockSpec((1, CH), lambda i: (0, 0)),
        ],
        out_specs=[
            pl.BlockSpec((2, RB, CH), lambda i: (0, i, 0)),
            pl.BlockSpec((1, CH), lambda i: (0, 0)),
        ],
        out_shape=[
            jax.ShapeDtypeStruct((2, N, CH), jnp.float32),
            jax.ShapeDtypeStruct((1, CH), jnp.float32),
        ],
    )(p3, a_tab3, b_tab3, gmax, fc_w, fc_b, q)


def _blend_body(z_ref, sc_ref, o_ref):
    srow = sc_ref[...]
    lane = lax.broadcasted_iota(jnp.int32, (1, CH), 1)
    s0 = jnp.sum(jnp.where(lane == 0, srow, 0.0))
    s1 = jnp.sum(jnp.where(lane == 1, srow, 0.0))
    mx = jnp.maximum(s0, s1)
    e0 = jnp.exp(s0 - mx)
    e1 = jnp.exp(s1 - mx)
    v0 = e0 / (e0 + e1)
    o_ref[...] = v0 * z_ref[0] + (1.0 - v0) * z_ref[1]


def _blend(z, scores):
    return pl.pallas_call(
        _blend_body,
        grid=(NBLK,),
        in_specs=[
            pl.BlockSpec((2, RB, CH), lambda i: (0, i, 0)),
            pl.BlockSpec((1, CH), lambda i: (0, 0)),
        ],
        out_specs=pl.BlockSpec((RB, CH), lambda i: (i, 0)),
        out_shape=jax.ShapeDtypeStruct((N, CH), jnp.float32),
    )(z, scores)


def kernel(x_paper, edge_index_cites, edge_index_refs, W_proj, att0, att1,
           fc_w, fc_b, q):
    a_tab3, b_tab3, gmax = _prep(x_paper, W_proj, att0, att1)
    a_tab = a_tab3.reshape(2 * N, AW)
    b_tab = b_tab3.reshape(2 * N, BW)
    ei = jnp.stack([edge_index_cites, edge_index_refs])
    off = jnp.array([0, N], jnp.int32).reshape(2, 1, 1)
    idx_tab = jnp.stack([ei[:, 0] + off[..., 0], ei[:, 1] + off[..., 0],
                         ei[:, 1]], axis=1)
    p = _edge_phase(a_tab, b_tab, idx_tab, gmax)
    p3 = p.reshape(2, ACCN, PW)[:, :N, :]
    z, scores = _finish(p3, a_tab3, b_tab3, gmax, fc_w,
                        fc_b.reshape(1, CH), q.reshape(1, CH))
    return _blend(z, scores)


# speedup vs baseline: 1.1837x; 1.0138x over previous
import functools

import jax
import jax.numpy as jnp
from jax import lax
from jax.experimental import pallas as pl
from jax.experimental.pallas import tpu as pltpu
from jax.experimental.pallas import tpu_sc as plsc

N = 10000
E = 320000
CH = 128
H = 8
D = 16
AW = 32
BW = 16
PW = 144
NBLK = 10
RB = N // NBLK

NSUB = 16
EPT = E // NSUB
CHUNK = 80
NCHUNK = EPT // CHUNK
ACCN = 10240
RPT = ACCN // NSUB
ZR = 32


def _leaky(t):
    return jnp.where(t >= 0, t, 0.2 * t)


def _prep_body(x_ref, w_ref, att0_ref, att1_ref,
               a_ref, b_ref, gmax_ref, gs_src, gs_dst):
    i = pl.program_id(0)
    hb = lax.dot_general(x_ref[...], w_ref[...], (((1,), (1,)), ((), ())),
                         preferred_element_type=jnp.float32)
    rows_s = []
    rows_d = []
    for m, att_ref in enumerate((att0_ref, att1_ref)):
        att = att_ref[...]
        a_dst = att[:, :D]
        a_src = att[:, D:]
        ss = lax.dot_general(hb, a_src, (((1,), (1,)), ((), ())),
                             preferred_element_type=jnp.float32)
        sd = lax.dot_general(hb, a_dst, (((1,), (1,)), ((), ())),
                             preferred_element_type=jnp.float32)
        a_ref[m] = jnp.concatenate([hb, ss, ss], axis=-1)
        b_ref[m] = jnp.concatenate([sd, sd], axis=-1)
        ms = jnp.max(ss, axis=0)
        md = jnp.max(sd, axis=0)
        rows_s.append(jnp.concatenate([ms, ms]))
        rows_d.append(jnp.concatenate([md, md]))
    sstack = jnp.stack(rows_s)
    dstack = jnp.stack(rows_d)

    @pl.when(i == 0)
    def _():
        gs_src[...] = sstack
        gs_dst[...] = dstack

    @pl.when(i > 0)
    def _():
        gs_src[...] = jnp.maximum(gs_src[...], sstack)
        gs_dst[...] = jnp.maximum(gs_dst[...], dstack)

    @pl.when(i == NBLK - 1)
    def _():
        gmax_ref[...] = _leaky(gs_src[...] + gs_dst[...])


def _prep(x, w, att0, att1):
    return pl.pallas_call(
        _prep_body,
        grid=(NBLK,),
        in_specs=[
            pl.BlockSpec((RB, CH), lambda i: (i, 0)),
            pl.BlockSpec((D, CH), lambda i: (0, 0)),
            pl.BlockSpec((H, 2 * D), lambda i: (0, 0)),
            pl.BlockSpec((H, 2 * D), lambda i: (0, 0)),
        ],
        out_specs=[
            pl.BlockSpec((2, RB, AW), lambda i: (0, i, 0)),
            pl.BlockSpec((2, RB, BW), lambda i: (0, i, 0)),
            pl.BlockSpec((2, 16), lambda i: (0, 0)),
        ],
        out_shape=[
            jax.ShapeDtypeStruct((2, ACCN, AW), jnp.float32),
            jax.ShapeDtypeStruct((2, ACCN, BW), jnp.float32),
            jax.ShapeDtypeStruct((2, 16), jnp.float32),
        ],
        scratch_shapes=[
            pltpu.VMEM((2, 16), jnp.float32),
            pltpu.VMEM((2, 16), jnp.float32),
        ],
    )(x, w, att0, att1)


def _edge_body(a_hbm, b_hbm, idx_hbm, gmax_hbm, out_hbm,
               idx_all, idx_sc, a_rows, b_rows, payload,
               gmax_v, zbuf, acc, sem_i, sem_a, sem_b, sem_p):
    c = lax.axis_index("c")
    s = lax.axis_index("s")

    pltpu.sync_copy(gmax_hbm.at[c], gmax_v)
    gv = gmax_v[...]

    def _zrow(r, _):
        for k in range(PW // 16):
            zbuf[r, pl.ds(k * 16, 16)] = jnp.zeros((16,), jnp.float32)
        return 0
    lax.fori_loop(0, ZR, _zrow, 0)
    for j in range(RPT // ZR):
        pltpu.sync_copy(zbuf, acc.at[pl.ds(s * RPT + j * ZR, ZR)])
    plsc.subcore_barrier()

    ebase = s * EPT
    co = c * ACCN
    last = NCHUNK - 1

    def _issue_idx(j, b):
        off = ebase + jnp.minimum(j, last) * CHUNK
        pltpu.async_copy(idx_hbm.at[c, :, pl.ds(off, CHUNK)], idx_all.at[b],
                         sem_i.at[b])

    def _wait_idx(b):
        pltpu.make_async_copy(idx_hbm.at[0, :, pl.ds(0, CHUNK)],
                              idx_all.at[b], sem_i.at[b]).wait()

    def _gather(b):
        pltpu.async_copy(a_hbm.at[idx_all.at[b, 0]], a_rows.at[b],
                         sem_a.at[b])
        pltpu.async_copy(b_hbm.at[idx_all.at[b, 1]], b_rows.at[b],
                         sem_b.at[b])

    def _wait_gather(b):
        pltpu.make_async_copy(a_hbm.at[idx_all.at[b, 0]], a_rows.at[b],
                              sem_a.at[b]).wait()
        pltpu.make_async_copy(b_hbm.at[idx_all.at[b, 1]], b_rows.at[b],
                              sem_b.at[b]).wait()

    def _compute(b):
        @plsc.parallel_loop(0, CHUNK, 1, unroll=4)
        def _edge(e):
            va = a_rows[b, e, pl.ds(0, 16)]
            vs = a_rows[b, e, pl.ds(16, 16)]
            vb = b_rows[b, e, pl.ds(0, 16)]
            ex = jnp.exp(_leaky(vs + vb) - gv)
            payload[b, e, pl.ds(CH, 16)] = ex
            for hh in range(H):
                w = ex.at[jnp.full((16,), hh, jnp.int32)].get(
                    mode="promise_in_bounds")
                payload[b, e, pl.ds(hh * 16, 16)] = w * va

    def _wait_scatter(b):
        pltpu.make_async_copy(payload.at[b], acc.at[idx_sc.at[b]],
                              sem_p.at[b]).wait()

    _issue_idx(0, 0)
    _wait_idx(0)
    _gather(0)
    _issue_idx(1, 1)

    def _two_chunks(j2, _):
        j = 2 * j2
        for b in (0, 1):
            nb = 1 - b
            _wait_gather(b)
            _wait_idx(nb)
            _gather(nb)

            @pl.when(j2 > 0)
            def _():
                _wait_scatter(b)
            for k in range(CHUNK // 16):
                sl = pl.ds(k * 16, 16)
                idx_sc[b, sl] = idx_all[b, 2, sl]
            _issue_idx(j + b + 2, b)
            _compute(b)
            pltpu.async_copy(payload.at[b], acc.at[idx_sc.at[b]],
                             sem_p.at[b], add=True)
        return 0

    lax.fori_loop(0, NCHUNK // 2, _two_chunks, 0)
    _wait_gather(0)
    _wait_idx(1)
    _wait_scatter(0)
    _wait_scatter(1)
    plsc.subcore_barrier()

    for j in range(RPT // ZR):
        r = s * RPT + j * ZR
        pltpu.sync_copy(acc.at[pl.ds(r, ZR)], out_hbm.at[pl.ds(co + r, ZR)])


def _edge_phase(a_tab, b_tab, idx_tab, gmax):
    f = pl.kernel(
        _edge_body,
        out_type=jax.ShapeDtypeStruct((2 * ACCN, PW), jnp.float32),
        mesh=plsc.VectorSubcoreMesh(core_axis_name="c", subcore_axis_name="s"),
        scratch_types=[
            pltpu.VMEM((2, 3, CHUNK), jnp.int32),
            pltpu.VMEM((2, CHUNK), jnp.int32),
            pltpu.VMEM((2, CHUNK, AW), jnp.float32),
            pltpu.VMEM((2, CHUNK, BW), jnp.float32),
            pltpu.VMEM((2, CHUNK, PW), jnp.float32),
            pltpu.VMEM((16,), jnp.float32),
            pltpu.VMEM((ZR, PW), jnp.float32),
            pltpu.VMEM_SHARED((ACCN, PW), jnp.float32),
            pltpu.SemaphoreType.DMA((2,)),
            pltpu.SemaphoreType.DMA((2,)),
            pltpu.SemaphoreType.DMA((2,)),
            pltpu.SemaphoreType.DMA((2,)),
        ],
        compiler_params=pltpu.CompilerParams(use_tc_tiling_on_sc=False),
    )
    return f(a_tab, b_tab, idx_tab, gmax)


RB2 = 1024
NBLK2 = ACCN // RB2


def _finish_body(p_ref, a_ref, b_ref, gmax_ref, fcw_ref, fcb_ref, q_ref,
                 o_ref, z_scr, sc_scr):
    ph = pl.program_id(0)
    i = pl.program_id(1)

    @pl.when(ph == 0)
    def _():
        rows = i * RB2 + lax.broadcasted_iota(jnp.int32, (RB2, 1), 0)
        rmask = rows < N
        parts = []
        for m in range(2):
            pm = p_ref[m]
            accf = pm[:, :CH]
            den = pm[:, CH:CH + H]
            hb = a_ref[m][:, :D]
            ss = a_ref[m][:, D:D + H]
            sd = b_ref[m][:, :H]
            g = gmax_ref[...][m, :H]
            exs = jnp.exp(_leaky(ss + sd) - g[None, :])
            den2 = den + exs + 1e-16
            exw = jnp.repeat(exs, D, axis=1)
            denw = jnp.repeat(den2, D, axis=1)
            hw = jnp.tile(hb, (1, H))
            out = (accf + exw * hw) / denw
            elu = jnp.where(out > 0, out,
                            jnp.exp(jnp.minimum(out, 0.0)) - 1.0)
            z_scr[m, pl.ds(i * RB2, RB2), :] = elu
            t = jnp.tanh(lax.dot_general(elu, fcw_ref[...],
                                         (((1,), (1,)), ((), ())),
                                         preferred_element_type=jnp.float32)
                         + fcb_ref[...])
            tq = jnp.where(rmask, t * q_ref[...], 0.0)
            parts.append(jnp.sum(tq) * (1.0 / N))
        lane = lax.broadcasted_iota(jnp.int32, (1, CH), 1)
        srow = jnp.where(lane == 0, parts[0],
                         jnp.where(lane == 1, parts[1], 0.0))

        @pl.when(i == 0)
        def _():
            sc_scr[...] = srow

        @pl.when(i > 0)
        def _():
            sc_scr[...] = sc_scr[...] + srow

    @pl.when(ph == 1)
    def _():
        srow = sc_scr[...]
        lane = lax.broadcasted_iota(jnp.int32, (1, CH), 1)
        s0 = jnp.sum(jnp.where(lane == 0, srow, 0.0))
        s1 = jnp.sum(jnp.where(lane == 1, srow, 0.0))
        mx = jnp.maximum(s0, s1)
        e0 = jnp.exp(s0 - mx)
        e1 = jnp.exp(s1 - mx)
        v0 = e0 / (e0 + e1)
        o_ref[...] = (v0 * z_scr[0, pl.ds(i * RB2, RB2), :]
                      + (1.0 - v0) * z_scr[1, pl.ds(i * RB2, RB2), :])


def _finish(p3, a_tab3, b_tab3, gmax, fc_w, fc_b, q):
    return pl.pallas_call(
        _finish_body,
        grid=(2, NBLK2),
        in_specs=[
            pl.BlockSpec((2, RB2, PW), lambda p, i: (0, i, 0)),
            pl.BlockSpec((2, RB2, AW), lambda p, i: (0, i, 0)),
            pl.BlockSpec((2, RB2, BW), lambda p, i: (0, i, 0)),
            pl.BlockSpec((2, 16), lambda p, i: (0, 0)),
            pl.BlockSpec((CH, CH), lambda p, i: (0, 0)),
            pl.BlockSpec((1, CH), lambda p, i: (0, 0)),
            pl.BlockSpec((1, CH), lambda p, i: (0, 0)),
        ],
        out_specs=pl.BlockSpec((RB2, CH), lambda p, i: (i, 0)),
        out_shape=jax.ShapeDtypeStruct((N, CH), jnp.float32),
        scratch_shapes=[
            pltpu.VMEM((2, ACCN, CH), jnp.float32),
            pltpu.VMEM((1, CH), jnp.float32),
        ],
    )(p3, a_tab3, b_tab3, gmax, fc_w, fc_b, q)


def kernel(x_paper, edge_index_cites, edge_index_refs, W_proj, att0, att1,
           fc_w, fc_b, q):
    a_tab3, b_tab3, gmax = _prep(x_paper, W_proj, att0, att1)
    a_tab = a_tab3.reshape(2 * ACCN, AW)
    b_tab = b_tab3.reshape(2 * ACCN, BW)
    ei = jnp.stack([edge_index_cites, edge_index_refs])
    off = jnp.array([0, ACCN], jnp.int32).reshape(2, 1)
    idx_tab = jnp.stack([ei[:, 0] + off, ei[:, 1] + off, ei[:, 1]],
                        axis=1)
    p = _edge_phase(a_tab, b_tab, idx_tab, gmax)
    return _finish(p.reshape(2, ACCN, PW), a_tab3, b_tab3, gmax, fc_w,
                   fc_b.reshape(1, CH), q.reshape(1, CH))


# speedup vs baseline: 1.5056x; 1.2719x over previous
import functools

import jax
import jax.numpy as jnp
from jax import lax
from jax.experimental import pallas as pl
from jax.experimental.pallas import tpu as pltpu
from jax.experimental.pallas import tpu_sc as plsc

N = 10000
E = 320000
CH = 128
H = 8
D = 16
AW = 32
BW = 16
PW = 144
NBLK = 10
RB = N // NBLK

NSUB = 16
EPT = E // NSUB
CHUNK = 80
NCHUNK = EPT // CHUNK
ACCN = 10240
RPT = ACCN // NSUB
ZR = 32


def _leaky(t):
    return jnp.where(t >= 0, t, 0.2 * t)


def _sel(rows, cols, fn):
    r = lax.broadcasted_iota(jnp.int32, (rows, cols), 0)
    c = lax.broadcasted_iota(jnp.int32, (rows, cols), 1)
    return fn(r, c).astype(jnp.float32)


def _prep_body(x_ref, w_ref, att0_ref, att1_ref,
               a_ref, b_ref, gmax_ref, gs_src, gs_dst):
    i = pl.program_id(0)
    hb = lax.dot_general(x_ref[...], w_ref[...], (((1,), (1,)), ((), ())),
                         preferred_element_type=jnp.float32)
    ph = _sel(D, AW, lambda r, c: r == c)
    ps = _sel(H, AW, lambda r, c: (c >= D) & (jnp.mod(c - D, H) == r))
    pb = _sel(H, BW, lambda r, c: jnp.mod(c, H) == r)
    a_h = lax.dot_general(hb, ph, (((1,), (0,)), ((), ())),
                          preferred_element_type=jnp.float32)
    for m, att_ref in enumerate((att0_ref, att1_ref)):
        att = att_ref[...]
        a_dst = att[:, :D]
        a_src = att[:, D:]
        ss = lax.dot_general(hb, a_src, (((1,), (1,)), ((), ())),
                             preferred_element_type=jnp.float32)
        sd = lax.dot_general(hb, a_dst, (((1,), (1,)), ((), ())),
                             preferred_element_type=jnp.float32)
        a_ref[m] = a_h + lax.dot_general(ss, ps, (((1,), (0,)), ((), ())),
                                         preferred_element_type=jnp.float32)
        sd16 = lax.dot_general(sd, pb, (((1,), (0,)), ((), ())),
                               preferred_element_type=jnp.float32)
        ss16 = lax.dot_general(ss, pb, (((1,), (0,)), ((), ())),
                               preferred_element_type=jnp.float32)
        b_ref[m] = sd16
        ms = jnp.max(ss16, axis=0, keepdims=True)
        md = jnp.max(sd16, axis=0, keepdims=True)
        sl = pl.ds(m, 1)

        @pl.when(i == 0)
        def _():
            gs_src[sl] = ms
            gs_dst[sl] = md

        @pl.when(i > 0)
        def _():
            gs_src[sl] = jnp.maximum(gs_src[sl], ms)
            gs_dst[sl] = jnp.maximum(gs_dst[sl], md)

    @pl.when(i == NBLK - 1)
    def _():
        gmax_ref[...] = _leaky(gs_src[...] + gs_dst[...])


def _prep(x, w, att0, att1):
    return pl.pallas_call(
        _prep_body,
        grid=(NBLK,),
        in_specs=[
            pl.BlockSpec((RB, CH), lambda i: (i, 0)),
            pl.BlockSpec((D, CH), lambda i: (0, 0)),
            pl.BlockSpec((H, 2 * D), lambda i: (0, 0)),
            pl.BlockSpec((H, 2 * D), lambda i: (0, 0)),
        ],
        out_specs=[
            pl.BlockSpec((2, RB, AW), lambda i: (0, i, 0)),
            pl.BlockSpec((2, RB, BW), lambda i: (0, i, 0)),
            pl.BlockSpec((2, 16), lambda i: (0, 0)),
        ],
        out_shape=[
            jax.ShapeDtypeStruct((2, ACCN, AW), jnp.float32),
            jax.ShapeDtypeStruct((2, ACCN, BW), jnp.float32),
            jax.ShapeDtypeStruct((2, 16), jnp.float32),
        ],
        scratch_shapes=[
            pltpu.VMEM((2, 16), jnp.float32),
            pltpu.VMEM((2, 16), jnp.float32),
        ],
    )(x, w, att0, att1)


def _edge_body(a_hbm, b_hbm, idx_hbm, gmax_hbm, out_hbm,
               idx_all, idx_sc, a_rows, b_rows, payload,
               gmax_v, zbuf, acc, sem_i, sem_a, sem_b, sem_p):
    c = lax.axis_index("c")
    s = lax.axis_index("s")

    pltpu.sync_copy(gmax_hbm.at[c], gmax_v)
    gv = gmax_v[...]

    def _zrow(r, _):
        for k in range(PW // 16):
            zbuf[r, pl.ds(k * 16, 16)] = jnp.zeros((16,), jnp.float32)
        return 0
    lax.fori_loop(0, ZR, _zrow, 0)
    for j in range(RPT // ZR):
        pltpu.sync_copy(zbuf, acc.at[pl.ds(s * RPT + j * ZR, ZR)])
    plsc.subcore_barrier()

    ebase = s * EPT
    co = c * ACCN
    last = NCHUNK - 1

    def _issue_idx(j, b):
        off = ebase + jnp.minimum(j, last) * CHUNK
        pltpu.async_copy(idx_hbm.at[c, :, pl.ds(off, CHUNK)], idx_all.at[b],
                         sem_i.at[b])

    def _wait_idx(b):
        pltpu.make_async_copy(idx_hbm.at[0, :, pl.ds(0, CHUNK)],
                              idx_all.at[b], sem_i.at[b]).wait()

    def _gather(b):
        pltpu.async_copy(a_hbm.at[idx_all.at[b, 0]], a_rows.at[b],
                         sem_a.at[b])
        pltpu.async_copy(b_hbm.at[idx_all.at[b, 1]], b_rows.at[b],
                         sem_b.at[b])

    def _wait_gather(b):
        pltpu.make_async_copy(a_hbm.at[idx_all.at[b, 0]], a_rows.at[b],
                              sem_a.at[b]).wait()
        pltpu.make_async_copy(b_hbm.at[idx_all.at[b, 1]], b_rows.at[b],
                              sem_b.at[b]).wait()

    def _compute(b):
        @plsc.parallel_loop(0, CHUNK, 1, unroll=4)
        def _edge(e):
            va = a_rows[b, e, pl.ds(0, 16)]
            vs = a_rows[b, e, pl.ds(16, 16)]
            vb = b_rows[b, e, pl.ds(0, 16)]
            ex = jnp.exp(_leaky(vs + vb) - gv)
            payload[b, e, pl.ds(CH, 16)] = ex
            for hh in range(H):
                w = ex.at[jnp.full((16,), hh, jnp.int32)].get(
                    mode="promise_in_bounds")
                payload[b, e, pl.ds(hh * 16, 16)] = w * va

    def _wait_scatter(b):
        pltpu.make_async_copy(payload.at[b], acc.at[idx_sc.at[b]],
                              sem_p.at[b]).wait()

    _issue_idx(0, 0)
    _wait_idx(0)
    _gather(0)
    _issue_idx(1, 1)

    def _two_chunks(j2, _):
        j = 2 * j2
        for b in (0, 1):
            nb = 1 - b
            _wait_gather(b)
            _wait_idx(nb)
            _gather(nb)

            @pl.when(j2 > 0)
            def _():
                _wait_scatter(b)
            for k in range(CHUNK // 16):
                sl = pl.ds(k * 16, 16)
                idx_sc[b, sl] = idx_all[b, 2, sl]
            _issue_idx(j + b + 2, b)
            _compute(b)
            pltpu.async_copy(payload.at[b], acc.at[idx_sc.at[b]],
                             sem_p.at[b], add=True)
        return 0

    lax.fori_loop(0, NCHUNK // 2, _two_chunks, 0)
    _wait_gather(0)
    _wait_idx(1)
    _wait_scatter(0)
    _wait_scatter(1)
    plsc.subcore_barrier()

    for j in range(RPT // ZR):
        r = s * RPT + j * ZR
        pltpu.sync_copy(acc.at[pl.ds(r, ZR)], out_hbm.at[pl.ds(co + r, ZR)])


def _edge_phase(a_tab, b_tab, idx_tab, gmax):
    f = pl.kernel(
        _edge_body,
        out_type=jax.ShapeDtypeStruct((2 * ACCN, PW), jnp.float32),
        mesh=plsc.VectorSubcoreMesh(core_axis_name="c", subcore_axis_name="s"),
        scratch_types=[
            pltpu.VMEM((2, 3, CHUNK), jnp.int32),
            pltpu.VMEM((2, CHUNK), jnp.int32),
            pltpu.VMEM((2, CHUNK, AW), jnp.float32),
            pltpu.VMEM((2, CHUNK, BW), jnp.float32),
            pltpu.VMEM((2, CHUNK, PW), jnp.float32),
            pltpu.VMEM((16,), jnp.float32),
            pltpu.VMEM((ZR, PW), jnp.float32),
            pltpu.VMEM_SHARED((ACCN, PW), jnp.float32),
            pltpu.SemaphoreType.DMA((2,)),
            pltpu.SemaphoreType.DMA((2,)),
            pltpu.SemaphoreType.DMA((2,)),
            pltpu.SemaphoreType.DMA((2,)),
        ],
        compiler_params=pltpu.CompilerParams(use_tc_tiling_on_sc=False),
    )
    return f(a_tab, b_tab, idx_tab, gmax)


RB2 = 1024
NBLK2 = ACCN // RB2


def _finish_body(p_ref, a_ref, b_ref, gmax_ref, fcw_ref, fcb_ref, q_ref,
                 o_ref, z_scr, sc_scr):
    ph = pl.program_id(0)
    i = pl.program_id(1)

    @pl.when(ph == 0)
    def _():
        rows = i * RB2 + lax.broadcasted_iota(jnp.int32, (RB2, 1), 0)
        rmask = rows < N
        wexp = _sel(H, CH, lambda r, c: (c // D) == r)
        wtile = _sel(D, CH, lambda r, c: jnp.mod(c, D) == r)
        parts = []
        for m in range(2):
            pm = p_ref[m]
            accf = pm[:, :CH]
            den = pm[:, CH:CH + H]
            hb = a_ref[m][:, :D]
            ss = a_ref[m][:, D:D + H]
            sd = b_ref[m][:, :H]
            g = gmax_ref[...][m, :H]
            exs = jnp.exp(_leaky(ss + sd) - g[None, :])
            rden = 1.0 / (den + exs + 1e-16)
            exw = lax.dot_general(exs, wexp, (((1,), (0,)), ((), ())),
                                  preferred_element_type=jnp.float32)
            rdw = lax.dot_general(rden, wexp, (((1,), (0,)), ((), ())),
                                  preferred_element_type=jnp.float32)
            hw = lax.dot_general(hb, wtile, (((1,), (0,)), ((), ())),
                                 preferred_element_type=jnp.float32)
            out = (accf + exw * hw) * rdw
            elu = jnp.where(out > 0, out,
                            jnp.exp(jnp.minimum(out, 0.0)) - 1.0)
            z_scr[m, pl.ds(i * RB2, RB2), :] = elu
            t = jnp.tanh(lax.dot_general(elu, fcw_ref[...],
                                         (((1,), (1,)), ((), ())),
                                         preferred_element_type=jnp.float32)
                         + fcb_ref[...])
            tq = jnp.where(rmask, t * q_ref[...], 0.0)
            parts.append(jnp.sum(tq) * (1.0 / N))
        lane = lax.broadcasted_iota(jnp.int32, (1, CH), 1)
        srow = jnp.where(lane == 0, parts[0],
                         jnp.where(lane == 1, parts[1], 0.0))

        @pl.when(i == 0)
        def _():
            sc_scr[...] = srow

        @pl.when(i > 0)
        def _():
            sc_scr[...] = sc_scr[...] + srow

    @pl.when(ph == 1)
    def _():
        srow = sc_scr[...]
        lane = lax.broadcasted_iota(jnp.int32, (1, CH), 1)
        s0 = jnp.sum(jnp.where(lane == 0, srow, 0.0))
        s1 = jnp.sum(jnp.where(lane == 1, srow, 0.0))
        mx = jnp.maximum(s0, s1)
        e0 = jnp.exp(s0 - mx)
        e1 = jnp.exp(s1 - mx)
        v0 = e0 / (e0 + e1)
        o_ref[...] = (v0 * z_scr[0, pl.ds(i * RB2, RB2), :]
                      + (1.0 - v0) * z_scr[1, pl.ds(i * RB2, RB2), :])


def _finish(p3, a_tab3, b_tab3, gmax, fc_w, fc_b, q):
    return pl.pallas_call(
        _finish_body,
        grid=(2, NBLK2),
        in_specs=[
            pl.BlockSpec((2, RB2, PW), lambda p, i: (0, i, 0)),
            pl.BlockSpec((2, RB2, AW), lambda p, i: (0, i, 0)),
            pl.BlockSpec((2, RB2, BW), lambda p, i: (0, i, 0)),
            pl.BlockSpec((2, 16), lambda p, i: (0, 0)),
            pl.BlockSpec((CH, CH), lambda p, i: (0, 0)),
            pl.BlockSpec((1, CH), lambda p, i: (0, 0)),
            pl.BlockSpec((1, CH), lambda p, i: (0, 0)),
        ],
        out_specs=pl.BlockSpec((RB2, CH), lambda p, i: (i, 0)),
        out_shape=jax.ShapeDtypeStruct((N, CH), jnp.float32),
        scratch_shapes=[
            pltpu.VMEM((2, ACCN, CH), jnp.float32),
            pltpu.VMEM((1, CH), jnp.float32),
        ],
    )(p3, a_tab3, b_tab3, gmax, fc_w, fc_b, q)


def kernel(x_paper, edge_index_cites, edge_index_refs, W_proj, att0, att1,
           fc_w, fc_b, q):
    a_tab3, b_tab3, gmax = _prep(x_paper, W_proj, att0, att1)
    a_tab = a_tab3.reshape(2 * ACCN, AW)
    b_tab = b_tab3.reshape(2 * ACCN, BW)
    ei = jnp.stack([edge_index_cites, edge_index_refs])
    off = jnp.array([0, ACCN], jnp.int32).reshape(2, 1)
    idx_tab = jnp.stack([ei[:, 0] + off, ei[:, 1] + off, ei[:, 1]],
                        axis=1)
    p = _edge_phase(a_tab, b_tab, idx_tab, gmax)
    return _finish(p.reshape(2, ACCN, PW), a_tab3, b_tab3, gmax, fc_w,
                   fc_b.reshape(1, CH), q.reshape(1, CH))
